# baseline (speedup 1.0000x reference)
import numpy as np
import jax
import jax.numpy as jnp
from jax.experimental import pallas as pl
from jax.experimental.pallas import tpu as pltpu

_B = 32
_L = 4096
_F = 64
_K = 2048
_H = 8192
_P = 126
_PPAD = 128

_KBLK = 256
_HBLK = 256
_NA = _K // _KBLK
_NB = _H // _HBLK

_l_idx = np.arange(_L, dtype=np.int64)[:, None]
_k_idx = np.arange(1, _K + 1, dtype=np.int64)[None, :]
_ang = (2.0 * np.pi / _L) * ((_l_idx * _k_idx) % _L).astype(np.float64)
_SCALE = 1.0 / np.sqrt(_L)
_DFT_C = np.ascontiguousarray((np.cos(_ang) * _SCALE).astype(np.float32))
_DFT_S = np.ascontiguousarray((-np.sin(_ang) * _SCALE).astype(np.float32))
del _l_idx, _k_idx, _ang


def _startfc_body(x_ref, w_ref, b_ref, o_ref):
    r = jax.lax.dot_general(x_ref[...], w_ref[...],
                            (((2,), (0,)), ((), ())),
                            preferred_element_type=jnp.float32)
    o_ref[...] = r + b_ref[...]


def _mega_body(xp_ref, c_ref, s_ref, w1_ref, b1_ref, w2_ref, b2_ref, wg_ref,
               o_ref, xs_ref, q0_ref, q1_ref):
    i = pl.program_id(0)

    @pl.when(i < _NA)
    def _dft():
        xp = xp_ref[...]
        xs_ref[0:_B, pl.ds(i * _KBLK, _KBLK)] = jnp.dot(
            xp, c_ref[...], preferred_element_type=jnp.float32)
        xs_ref[_B:2 * _B, pl.ds(i * _KBLK, _KBLK)] = jnp.dot(
            xp, s_ref[...], preferred_element_type=jnp.float32)

    @pl.when(i == 0)
    def _init():
        q0_ref[...] = jnp.zeros_like(q0_ref)
        q1_ref[...] = jnp.zeros_like(q1_ref)

    @pl.when(i >= _NA)
    def _mlp():
        xs = xs_ref[...]
        p0 = jnp.dot(xs, w1_ref[0], preferred_element_type=jnp.float32)
        p1 = jnp.dot(xs, w1_ref[1], preferred_element_type=jnp.float32)
        o1r = jnp.maximum(p0[0:_B] - p1[_B:2 * _B] + b1_ref[0:1, :], 0.0)
        o1i = jnp.maximum(p0[_B:2 * _B] + p1[0:_B] + b1_ref[1:2, :], 0.0)
        o1 = jnp.concatenate([o1r, o1i], axis=0)
        q0_ref[...] += jnp.dot(o1, w2_ref[0], preferred_element_type=jnp.float32)
        q1_ref[...] += jnp.dot(o1, w2_ref[1], preferred_element_type=jnp.float32)

    @pl.when(i == _NA + _NB - 1)
    def _fini():
        q0 = q0_ref[...]
        q1 = q1_ref[...]
        o2r = q0[0:_B] - q1[_B:2 * _B] + b2_ref[0:1, :]
        o2i = q0[_B:2 * _B] + q1[0:_B] + b2_ref[1:2, :]
        amp = jnp.sqrt(o2r * o2r + o2i * o2i)
        lg = jnp.dot(amp, wg_ref[...], preferred_element_type=jnp.float32)
        col = jax.lax.broadcasted_iota(jnp.int32, (_B, _PPAD), 1)
        neg = jnp.float32(-3e38)
        big = jnp.int32(1 << 30)
        lm = jnp.where(col < _P, lg, neg)
        m1 = jnp.max(lm, axis=1, keepdims=True)
        i1 = jnp.min(jnp.where(lm == m1, col, big), axis=1, keepdims=True)
        lm2 = jnp.where(col == i1, neg, lm)
        m2 = jnp.max(lm2, axis=1, keepdims=True)
        i2 = jnp.min(jnp.where(lm2 == m2, col, big), axis=1, keepdims=True)
        e = jnp.exp(m2 - m1)
        w1v = 1.0 / (1.0 + e)
        w2v = e / (1.0 + e)
        o_ref[...] = (jnp.where(col == i1, w1v, 0.0)
                      + jnp.where(col == i2, w2v, 0.0))


def kernel(x, training, start_w, start_b, w1, b1, w2, b2, w_gate):
    del training
    f32 = jnp.float32
    dft_c = jnp.asarray(_DFT_C)
    dft_s = jnp.asarray(_DFT_S)
    wg_pad = jnp.pad(w_gate, ((0, 0), (0, _PPAD - _P)))
    w_bd = jnp.kron(jnp.eye(8, dtype=f32), start_w)
    b2d = jnp.reshape(start_b, (1, 1, 1)).astype(f32)
    x_r = jnp.reshape(x, (_B, _L // 8, 8 * _F))

    xp8 = pl.pallas_call(
        _startfc_body,
        grid=(4,),
        in_specs=[
            pl.BlockSpec((8, _L // 8, 8 * _F), lambda i: (i, 0, 0)),
            pl.BlockSpec((8 * _F, 8), lambda i: (0, 0)),
            pl.BlockSpec((1, 1, 1), lambda i: (0, 0, 0)),
        ],
        out_specs=pl.BlockSpec((8, _L // 8, 8), lambda i: (i, 0, 0)),
        out_shape=jax.ShapeDtypeStruct((_B, _L // 8, 8), f32),
    )(x_r, w_bd, b2d)
    xp = jnp.reshape(xp8, (_B, _L))

    gates = pl.pallas_call(
        _mega_body,
        grid=(_NA + _NB,),
        in_specs=[
            pl.BlockSpec((_B, _L), lambda i: (0, 0)),
            pl.BlockSpec((_L, _KBLK), lambda i: (0, jnp.minimum(i, _NA - 1))),
            pl.BlockSpec((_L, _KBLK), lambda i: (0, jnp.minimum(i, _NA - 1))),
            pl.BlockSpec((2, _K, _HBLK),
                         lambda i: (0, 0, jnp.clip(i - _NA, 0, _NB - 1))),
            pl.BlockSpec((2, _HBLK),
                         lambda i: (0, jnp.clip(i - _NA, 0, _NB - 1))),
            pl.BlockSpec((2, _HBLK, _K),
                         lambda i: (0, jnp.clip(i - _NA, 0, _NB - 1), 0)),
            pl.BlockSpec((2, _K), lambda i: (0, 0)),
            pl.BlockSpec((_K, _PPAD), lambda i: (0, 0)),
        ],
        out_specs=pl.BlockSpec((_B, _PPAD), lambda i: (0, 0)),
        out_shape=jax.ShapeDtypeStruct((_B, _PPAD), f32),
        scratch_shapes=[
            pltpu.VMEM((2 * _B, _K), f32),
            pltpu.VMEM((2 * _B, _K), f32),
            pltpu.VMEM((2 * _B, _K), f32),
        ],
        compiler_params=pltpu.CompilerParams(
            dimension_semantics=("arbitrary",)),
    )(xp, dft_c, dft_s, w1, b1, w2, b2, wg_pad)

    return gates[:, :_P]
